# baseline (speedup 1.0000x reference)
import jax
import jax.numpy as jnp
import numpy as np
from jax import lax
from jax.experimental import pallas as pl
from jax.experimental.pallas import tpu as pltpu
from jax.experimental.pallas import tpu_sc as plsc

N = 10000
E = 320000
D = 128
NC = 2
NS = 16
NW = NC * NS
EC = 80
EPT = E // NW
KCH = EPT // EC
NWC = N // EC

_PERM = np.empty((D,), np.int32)
for _j in range(D // 32):
    for _i in range(16):
        _PERM[32 * _j + 2 * _i] = 32 * _j + _i
        _PERM[32 * _j + 2 * _i + 1] = 32 * _j + 16 + _i

_HI = np.int32(np.uint32(0xFFFF0000).view(np.int32))


def _tc_pre_body(x_ref, w_ref, w2_ref, as_ref, ad_ref,
                 xl_ref, xlp_ref, asad_ref, es_ref):
    xb = x_ref[...]
    xl = jnp.dot(xb, w_ref[...], preferred_element_type=jnp.float32)
    xl_ref[...] = xl
    xlp = jnp.dot(xb, w2_ref[...], preferred_element_type=jnp.float32)
    xlp_ref[...] = xlp.astype(jnp.bfloat16)
    sv = jnp.sum(xl * as_ref[...], axis=1, keepdims=True)
    dv = jnp.sum(xl * ad_ref[...], axis=1, keepdims=True)
    asad_ref[...] = jnp.concatenate(
        [sv.astype(jnp.bfloat16), dv.astype(jnp.bfloat16)], axis=1)
    z = sv + dv
    es_ref[...] = jnp.exp(jnp.maximum(z, 0.2 * z))


def _sc_body(xlp, asad, src, dst, pout, pden,
             asad_v, src0, src1, src2, src3, dst0, dst1, dst2, dst3, eav,
             rb0, rb1, rf0, rf1, acc_sh, den_sh,
             sg0, sg1, si0, si1, si2, si3, sem_s):
    c = lax.axis_index("c")
    s = lax.axis_index("s")
    w = s * NC + c
    zero16 = jnp.zeros((16,), jnp.float32)
    srcs = (src0, src1, src2, src3)
    dsts = (dst0, dst1, dst2, dst3)
    rowsb = (rb0, rb1)
    rows_f = (rf0, rf1)
    sem_g = (sg0, sg1)
    sem_i = (si0, si1, si2, si3)

    pltpu.sync_copy(asad, asad_v)

    def zs_body(r, _):
        for j in range(8):
            rf0[r, pl.ds(16 * j, 16)] = zero16
        return 0
    lax.fori_loop(0, EC, zs_body, 0)
    for g in range(EC // 16):
        eav[pl.ds(16 * g, 16)] = zero16

    def zacc_body(q, _):
        ch = s + q * NS

        @pl.when(ch < NWC)
        def _():
            pltpu.sync_copy(rf0, acc_sh.at[pl.ds(ch * EC, EC)])
            pltpu.sync_copy(eav, den_sh.at[pl.ds(ch * EC, EC)])
        return 0
    lax.fori_loop(0, -(-NWC // NS), zacc_body, 0)

    plsc.subcore_barrier()

    eb = w * EPT

    def fire_idx(k, m4):
        base = eb + k * EC
        pltpu.async_copy(src.at[pl.ds(base, EC)], srcs[m4], sem_i[m4])
        pltpu.async_copy(dst.at[pl.ds(base, EC)], dsts[m4], sem_i[m4])

    def drain_idx(m4):
        pltpu.make_async_copy(src.at[pl.ds(0, EC)], srcs[m4], sem_i[m4]).wait()
        pltpu.make_async_copy(dst.at[pl.ds(0, EC)], dsts[m4], sem_i[m4]).wait()

    def fire_gather(m4, m2):
        pltpu.async_copy(xlp.at[srcs[m4]], rowsb[m2], sem_g[m2])

    def drain_gather(m2):
        pltpu.make_async_copy(
            xlp.at[pl.ds(0, EC)], rowsb[m2], sem_g[m2]).wait()

    def step(k, m4, b, first=False, fi2=True, fg1=True):
        o = 1 - b
        drain_gather(b)
        for g in range(EC // 16):
            si = srcs[m4][pl.ds(16 * g, 16)]
            di = dsts[m4][pl.ds(16 * g, 16)]
            ps = plsc.load_gather(asad_v, [si])
            pd = plsc.load_gather(asad_v, [di])
            a1 = plsc.bitcast(jnp.left_shift(ps, 16), jnp.float32)
            d2 = plsc.bitcast(jnp.bitwise_and(pd, _HI), jnp.float32)
            z = a1 + d2
            eav[pl.ds(16 * g, 16)] = jnp.exp(jnp.maximum(z, 0.2 * z))
        if not first:
            pltpu.make_async_copy(
                pout.at[pl.ds(0, EC)], rows_f[o], sem_s).wait()
        if fi2:
            fire_idx(k + 2, (m4 + 2) % 4)
        if fg1:
            drain_idx((m4 + 1) % 4)
            fire_gather((m4 + 1) % 4, o)

        zv16 = lax.iota(jnp.int32, 16) * 0

        def scale_body(g, _):
            g16 = 16 * g
            for i in range(16):
                e = plsc.load_gather(eav, [zv16 + (g16 + i)])
                r = g16 + i
                for jj in range(4):
                    p = rowsb[b][r, pl.ds(16 * jj, 16)]
                    av = plsc.bitcast(jnp.left_shift(p, 16), jnp.float32)
                    bv = plsc.bitcast(jnp.bitwise_and(p, _HI), jnp.float32)
                    rows_f[b][r, pl.ds(32 * jj, 16)] = av * e
                    rows_f[b][r, pl.ds(32 * jj + 16, 16)] = bv * e
            return 0
        lax.fori_loop(0, EC // 16, scale_body, 0)

        pltpu.async_copy(rows_f[b], acc_sh.at[dsts[m4]], sem_s, add=True)
        pltpu.sync_copy(eav, den_sh.at[dsts[m4]], add=True)

    fire_idx(0, 0)
    fire_idx(1, 1)
    drain_idx(0)
    fire_gather(0, 0)
    step(0, 0, 0, first=True)

    def body4(j, _):
        k = 4 * j + 1
        step(k, 1, 1)
        step(k + 1, 2, 0)
        step(k + 2, 3, 1)
        step(k + 3, 0, 0)
        return 0
    lax.fori_loop(0, (KCH - 5) // 4, body4, 0)

    step(KCH - 4, 1, 1)
    step(KCH - 3, 2, 0)
    step(KCH - 2, 3, 1, fi2=False)
    step(KCH - 1, 0, 0, fi2=False, fg1=False)
    pltpu.make_async_copy(pout.at[pl.ds(0, EC)], rows_f[0], sem_s).wait()

    plsc.subcore_barrier()

    def wo_body(q, _):
        ch = s + q * NS

        @pl.when(ch < NWC)
        def _():
            r0 = ch * EC
            pltpu.sync_copy(acc_sh.at[pl.ds(r0, EC)], rf0)
            pltpu.sync_copy(rf0, pout.at[pl.ds(c * N + r0, EC)])
            pltpu.sync_copy(den_sh.at[pl.ds(r0, EC)], eav)
            pltpu.sync_copy(eav, pden.at[pl.ds(c * N + r0, EC)])
        return 0
    lax.fori_loop(0, -(-NWC // NS), wo_body, 0)


def _tc_post_body(p_ref, d_ref, xl_ref, x_ref, es_ref, b_ref, o_ref):
    es = es_ref[...]
    num = p_ref[0] + p_ref[1] + es * xl_ref[...]
    den = d_ref[0] + d_ref[1] + es + 1e-16
    o_ref[...] = num / den + b_ref[...] + x_ref[...]


_BR = 1000


@jax.jit
def kernel(x, edge_index, W, att_src, att_dst, bias):
    grid = (N // _BR,)
    W2 = W[:, _PERM]
    xl, xlp, asad, es = pl.pallas_call(
        _tc_pre_body,
        grid=grid,
        in_specs=[
            pl.BlockSpec((_BR, D), lambda i: (i, 0)),
            pl.BlockSpec((D, D), lambda i: (0, 0)),
            pl.BlockSpec((D, D), lambda i: (0, 0)),
            pl.BlockSpec((1, D), lambda i: (0, 0)),
            pl.BlockSpec((1, D), lambda i: (0, 0)),
        ],
        out_specs=[
            pl.BlockSpec((_BR, D), lambda i: (i, 0)),
            pl.BlockSpec((_BR, D), lambda i: (i, 0)),
            pl.BlockSpec((_BR, 2), lambda i: (i, 0)),
            pl.BlockSpec((_BR, 1), lambda i: (i, 0)),
        ],
        out_shape=[
            jax.ShapeDtypeStruct((N, D), jnp.float32),
            jax.ShapeDtypeStruct((N, D), jnp.bfloat16),
            jax.ShapeDtypeStruct((N, 2), jnp.bfloat16),
            jax.ShapeDtypeStruct((N, 1), jnp.float32),
        ],
    )(x, W, W2, att_src, att_dst)

    sc = pl.kernel(
        _sc_body,
        out_type=[
            jax.ShapeDtypeStruct((NC * N, D), jnp.float32),
            jax.ShapeDtypeStruct((NC * N,), jnp.float32),
        ],
        mesh=plsc.VectorSubcoreMesh(core_axis_name="c", subcore_axis_name="s"),
        compiler_params=pltpu.CompilerParams(
            needs_layout_passes=False, use_tc_tiling_on_sc=False),
        scratch_types=[
            pltpu.VMEM((N,), jnp.int32),
            pltpu.VMEM((EC,), jnp.int32),
            pltpu.VMEM((EC,), jnp.int32),
            pltpu.VMEM((EC,), jnp.int32),
            pltpu.VMEM((EC,), jnp.int32),
            pltpu.VMEM((EC,), jnp.int32),
            pltpu.VMEM((EC,), jnp.int32),
            pltpu.VMEM((EC,), jnp.int32),
            pltpu.VMEM((EC,), jnp.int32),
            pltpu.VMEM((EC,), jnp.float32),
            pltpu.VMEM((EC, D // 2), jnp.int32),
            pltpu.VMEM((EC, D // 2), jnp.int32),
            pltpu.VMEM((EC, D), jnp.float32),
            pltpu.VMEM((EC, D), jnp.float32),
            pltpu.VMEM_SHARED((N, D), jnp.float32),
            pltpu.VMEM_SHARED((N,), jnp.float32),
            pltpu.SemaphoreType.DMA,
            pltpu.SemaphoreType.DMA,
            pltpu.SemaphoreType.DMA,
            pltpu.SemaphoreType.DMA,
            pltpu.SemaphoreType.DMA,
            pltpu.SemaphoreType.DMA,
            pltpu.SemaphoreType.DMA,
        ],
    )
    xlp_i32 = jax.lax.bitcast_convert_type(
        xlp.reshape(N, D // 2, 2), jnp.int32)
    asad_i32 = jax.lax.bitcast_convert_type(
        asad.reshape(N, 1, 2), jnp.int32).reshape(N)
    pout, pden = sc(xlp_i32, asad_i32, edge_index[0], edge_index[1])

    out = pl.pallas_call(
        _tc_post_body,
        grid=grid,
        in_specs=[
            pl.BlockSpec((NC, _BR, D), lambda i: (0, i, 0)),
            pl.BlockSpec((NC, _BR, 1), lambda i: (0, i, 0)),
            pl.BlockSpec((_BR, D), lambda i: (i, 0)),
            pl.BlockSpec((_BR, D), lambda i: (i, 0)),
            pl.BlockSpec((_BR, 1), lambda i: (i, 0)),
            pl.BlockSpec((1, D), lambda i: (0, 0)),
        ],
        out_specs=pl.BlockSpec((_BR, D), lambda i: (i, 0)),
        out_shape=jax.ShapeDtypeStruct((N, D), jnp.float32),
    )(pout.reshape(NC, N, D), pden.reshape(NC, N, 1), xl, x, es,
      bias.reshape(1, D))
    return out


# speedup vs baseline: 1.7546x; 1.7546x over previous
import jax
import jax.numpy as jnp
import numpy as np
from jax import lax
from jax.experimental import pallas as pl
from jax.experimental.pallas import tpu as pltpu
from jax.experimental.pallas import tpu_sc as plsc

N = 10000
E = 320000
D = 128
NC = 2
NS = 16
NW = NC * NS
EC = 80
EPT = E // NW
KCH = EPT // EC
NWC = N // EC

_PERM = np.empty((D,), np.int32)
for _j in range(D // 32):
    for _i in range(16):
        _PERM[32 * _j + 2 * _i] = 32 * _j + _i
        _PERM[32 * _j + 2 * _i + 1] = 32 * _j + 16 + _i

_HI = np.int32(np.uint32(0xFFFF0000).view(np.int32))


def _tc_pre_body(x_ref, w_ref, w2_ref, as_ref, ad_ref,
                 xl_ref, xlp_ref, asad_ref, es_ref):
    xb = x_ref[...]
    xl = jnp.dot(xb, w_ref[...], preferred_element_type=jnp.float32)
    xl_ref[...] = xl
    xlp = jnp.dot(xb, w2_ref[...], preferred_element_type=jnp.float32)
    xlp_ref[...] = xlp.astype(jnp.bfloat16)
    sv = jnp.sum(xl * as_ref[...], axis=1, keepdims=True)
    dv = jnp.sum(xl * ad_ref[...], axis=1, keepdims=True)
    asad_ref[...] = jnp.concatenate(
        [sv.astype(jnp.bfloat16), dv.astype(jnp.bfloat16)], axis=1)
    z = sv + dv
    es_ref[...] = jnp.exp(jnp.maximum(z, 0.2 * z))


def _sc_body(xlp, asad, src, dst, pout, pden,
             asad_v, src0, src1, src2, src3, dst0, dst1, dst2, dst3, eav,
             rb0, rb1, rf0, rf1, acc_sh, den_sh,
             sg0, sg1, si0, si1, si2, si3, sem_s):
    c = lax.axis_index("c")
    s = lax.axis_index("s")
    w = s * NC + c
    zero16 = jnp.zeros((16,), jnp.float32)
    srcs = (src0, src1, src2, src3)
    dsts = (dst0, dst1, dst2, dst3)
    rowsb = (rb0, rb1)
    rows_f = (rf0, rf1)
    sem_g = (sg0, sg1)
    sem_i = (si0, si1, si2, si3)

    pltpu.sync_copy(asad, asad_v)

    def zs_body(r, _):
        for j in range(8):
            rf0[r, pl.ds(16 * j, 16)] = zero16
        return 0
    lax.fori_loop(0, EC, zs_body, 0)
    for g in range(EC // 16):
        eav[pl.ds(16 * g, 16)] = zero16

    def zacc_body(q, _):
        ch = s + q * NS

        @pl.when(ch < NWC)
        def _():
            pltpu.sync_copy(rf0, acc_sh.at[pl.ds(ch * EC, EC)])
            pltpu.sync_copy(eav, den_sh.at[pl.ds(ch * EC, EC)])
        return 0
    lax.fori_loop(0, -(-NWC // NS), zacc_body, 0)

    plsc.subcore_barrier()

    eb = w * EPT

    def fire_idx(k, m4):
        base = eb + k * EC
        pltpu.async_copy(src.at[pl.ds(base, EC)], srcs[m4], sem_i[m4])
        pltpu.async_copy(dst.at[pl.ds(base, EC)], dsts[m4], sem_i[m4])

    def drain_idx(m4):
        pltpu.make_async_copy(src.at[pl.ds(0, EC)], srcs[m4], sem_i[m4]).wait()
        pltpu.make_async_copy(dst.at[pl.ds(0, EC)], dsts[m4], sem_i[m4]).wait()

    def fire_gather(m4, m2):
        pltpu.async_copy(xlp.at[srcs[m4]], rowsb[m2], sem_g[m2])

    def drain_gather(m2):
        pltpu.make_async_copy(
            xlp.at[pl.ds(0, EC)], rowsb[m2], sem_g[m2]).wait()

    def step(k, m4, b, first=False, fi2=True, fg1=True):
        o = 1 - b
        drain_gather(b)
        for g in range(EC // 16):
            si = srcs[m4][pl.ds(16 * g, 16)]
            di = dsts[m4][pl.ds(16 * g, 16)]
            ps = plsc.load_gather(asad_v, [si])
            pd = plsc.load_gather(asad_v, [di])
            a1 = plsc.bitcast(jnp.left_shift(ps, 16), jnp.float32)
            d2 = plsc.bitcast(jnp.bitwise_and(pd, _HI), jnp.float32)
            z = a1 + d2
            eav[pl.ds(16 * g, 16)] = jnp.exp(jnp.maximum(z, 0.2 * z))
        if not first:
            pltpu.make_async_copy(
                pout.at[pl.ds(0, EC)], rows_f[o], sem_s).wait()
        if fi2:
            fire_idx(k + 2, (m4 + 2) % 4)
        if fg1:
            drain_idx((m4 + 1) % 4)
            fire_gather((m4 + 1) % 4, o)

        zv16 = lax.iota(jnp.int32, 16) * 0

        @plsc.parallel_loop(0, EC, 1, unroll=8)
        def _(r):
            e = plsc.load_gather(eav, [zv16 + r])
            for jj in range(4):
                p = rowsb[b][r, pl.ds(16 * jj, 16)]
                av = plsc.bitcast(jnp.left_shift(p, 16), jnp.float32)
                bv = plsc.bitcast(jnp.bitwise_and(p, _HI), jnp.float32)
                rows_f[b][r, pl.ds(32 * jj, 16)] = av * e
                rows_f[b][r, pl.ds(32 * jj + 16, 16)] = bv * e

        pltpu.async_copy(rows_f[b], acc_sh.at[dsts[m4]], sem_s, add=True)
        pltpu.sync_copy(eav, den_sh.at[dsts[m4]], add=True)

    fire_idx(0, 0)
    fire_idx(1, 1)
    drain_idx(0)
    fire_gather(0, 0)
    step(0, 0, 0, first=True)

    def body4(j, _):
        k = 4 * j + 1
        step(k, 1, 1)
        step(k + 1, 2, 0)
        step(k + 2, 3, 1)
        step(k + 3, 0, 0)
        return 0
    lax.fori_loop(0, (KCH - 5) // 4, body4, 0)

    step(KCH - 4, 1, 1)
    step(KCH - 3, 2, 0)
    step(KCH - 2, 3, 1, fi2=False)
    step(KCH - 1, 0, 0, fi2=False, fg1=False)
    pltpu.make_async_copy(pout.at[pl.ds(0, EC)], rows_f[0], sem_s).wait()

    plsc.subcore_barrier()

    def wo_body(q, _):
        ch = s + q * NS

        @pl.when(ch < NWC)
        def _():
            r0 = ch * EC
            pltpu.sync_copy(acc_sh.at[pl.ds(r0, EC)], rf0)
            pltpu.sync_copy(rf0, pout.at[pl.ds(c * N + r0, EC)])
            pltpu.sync_copy(den_sh.at[pl.ds(r0, EC)], eav)
            pltpu.sync_copy(eav, pden.at[pl.ds(c * N + r0, EC)])
        return 0
    lax.fori_loop(0, -(-NWC // NS), wo_body, 0)


def _tc_post_body(p_ref, d_ref, xl_ref, x_ref, es_ref, b_ref, o_ref):
    es = es_ref[...]
    num = p_ref[0] + p_ref[1] + es * xl_ref[...]
    den = d_ref[0] + d_ref[1] + es + 1e-16
    o_ref[...] = num / den + b_ref[...] + x_ref[...]


_BR = 1000


@jax.jit
def kernel(x, edge_index, W, att_src, att_dst, bias):
    grid = (N // _BR,)
    W2 = W[:, _PERM]
    xl, xlp, asad, es = pl.pallas_call(
        _tc_pre_body,
        grid=grid,
        in_specs=[
            pl.BlockSpec((_BR, D), lambda i: (i, 0)),
            pl.BlockSpec((D, D), lambda i: (0, 0)),
            pl.BlockSpec((D, D), lambda i: (0, 0)),
            pl.BlockSpec((1, D), lambda i: (0, 0)),
            pl.BlockSpec((1, D), lambda i: (0, 0)),
        ],
        out_specs=[
            pl.BlockSpec((_BR, D), lambda i: (i, 0)),
            pl.BlockSpec((_BR, D), lambda i: (i, 0)),
            pl.BlockSpec((_BR, 2), lambda i: (i, 0)),
            pl.BlockSpec((_BR, 1), lambda i: (i, 0)),
        ],
        out_shape=[
            jax.ShapeDtypeStruct((N, D), jnp.float32),
            jax.ShapeDtypeStruct((N, D), jnp.bfloat16),
            jax.ShapeDtypeStruct((N, 2), jnp.bfloat16),
            jax.ShapeDtypeStruct((N, 1), jnp.float32),
        ],
    )(x, W, W2, att_src, att_dst)

    sc = pl.kernel(
        _sc_body,
        out_type=[
            jax.ShapeDtypeStruct((NC * N, D), jnp.float32),
            jax.ShapeDtypeStruct((NC * N,), jnp.float32),
        ],
        mesh=plsc.VectorSubcoreMesh(core_axis_name="c", subcore_axis_name="s"),
        compiler_params=pltpu.CompilerParams(
            needs_layout_passes=False, use_tc_tiling_on_sc=False),
        scratch_types=[
            pltpu.VMEM((N,), jnp.int32),
            pltpu.VMEM((EC,), jnp.int32),
            pltpu.VMEM((EC,), jnp.int32),
            pltpu.VMEM((EC,), jnp.int32),
            pltpu.VMEM((EC,), jnp.int32),
            pltpu.VMEM((EC,), jnp.int32),
            pltpu.VMEM((EC,), jnp.int32),
            pltpu.VMEM((EC,), jnp.int32),
            pltpu.VMEM((EC,), jnp.int32),
            pltpu.VMEM((EC,), jnp.float32),
            pltpu.VMEM((EC, D // 2), jnp.int32),
            pltpu.VMEM((EC, D // 2), jnp.int32),
            pltpu.VMEM((EC, D), jnp.float32),
            pltpu.VMEM((EC, D), jnp.float32),
            pltpu.VMEM_SHARED((N, D), jnp.float32),
            pltpu.VMEM_SHARED((N,), jnp.float32),
            pltpu.SemaphoreType.DMA,
            pltpu.SemaphoreType.DMA,
            pltpu.SemaphoreType.DMA,
            pltpu.SemaphoreType.DMA,
            pltpu.SemaphoreType.DMA,
            pltpu.SemaphoreType.DMA,
            pltpu.SemaphoreType.DMA,
        ],
    )
    xlp_i32 = jax.lax.bitcast_convert_type(
        xlp.reshape(N, D // 2, 2), jnp.int32)
    asad_i32 = jax.lax.bitcast_convert_type(
        asad.reshape(N, 1, 2), jnp.int32).reshape(N)
    pout, pden = sc(xlp_i32, asad_i32, edge_index[0], edge_index[1])

    out = pl.pallas_call(
        _tc_post_body,
        grid=grid,
        in_specs=[
            pl.BlockSpec((NC, _BR, D), lambda i: (0, i, 0)),
            pl.BlockSpec((NC, _BR, 1), lambda i: (0, i, 0)),
            pl.BlockSpec((_BR, D), lambda i: (i, 0)),
            pl.BlockSpec((_BR, D), lambda i: (i, 0)),
            pl.BlockSpec((_BR, 1), lambda i: (i, 0)),
            pl.BlockSpec((1, D), lambda i: (0, 0)),
        ],
        out_specs=pl.BlockSpec((_BR, D), lambda i: (i, 0)),
        out_shape=jax.ShapeDtypeStruct((N, D), jnp.float32),
    )(pout.reshape(NC, N, D), pden.reshape(NC, N, 1), xl, x, es,
      bias.reshape(1, D))
    return out
